# baseline (speedup 1.0000x reference)
import jax
import jax.numpy as jnp
from jax.experimental import pallas as pl
from jax.experimental.pallas import tpu as pltpu


_HEAD = 640
_TAIL = 384


def _pool_kernel(x_ref, wp_ref, b_ref, w_hbm, o_ref, w_tail_v, sem):
    cp = pltpu.make_async_copy(
        w_hbm.at[pl.ds(_HEAD, _TAIL), :], w_tail_v, sem
    )
    cp.start()

    x = x_ref[:, 0, :]

    acc0 = jax.lax.dot_general(
        x, wp_ref[...],
        dimension_numbers=(((1,), (1,)), ((), ())),
        preferred_element_type=jnp.float32,
    )
    o_ref[:, pl.ds(0, _HEAD)] = jnp.tanh(acc0 + b_ref[:, pl.ds(0, _HEAD)])

    cp.wait()
    acc1 = jax.lax.dot_general(
        x, w_tail_v[...],
        dimension_numbers=(((1,), (1,)), ((), ())),
        preferred_element_type=jnp.float32,
    )
    o_ref[:, pl.ds(_HEAD, _TAIL)] = jnp.tanh(
        acc1 + b_ref[:, pl.ds(_HEAD, _TAIL)]
    )


@jax.jit
def kernel(hidden_states, W, b):
    B, _, H = hidden_states.shape
    O = W.shape[0]

    b2 = b.reshape(1, O)

    out = pl.pallas_call(
        _pool_kernel,
        grid=(1,),
        in_specs=[
            pl.BlockSpec((B, 8, H), lambda i: (0, 0, 0)),
            pl.BlockSpec((_HEAD, H), lambda i: (0, 0)),
            pl.BlockSpec((1, O), lambda i: (0, 0)),
            pl.BlockSpec(memory_space=pl.ANY),
        ],
        out_specs=pl.BlockSpec((B, O), lambda i: (0, 0)),
        out_shape=jax.ShapeDtypeStruct((B, O), jnp.float32),
        scratch_shapes=[
            pltpu.VMEM((_TAIL, H), jnp.float32),
            pltpu.SemaphoreType.DMA,
        ],
    )(hidden_states, W, b2, W)
    return out


# speedup vs baseline: 1.2712x; 1.2712x over previous
import jax
import jax.numpy as jnp
from jax.experimental import pallas as pl
from jax.experimental.pallas import tpu as pltpu


_CHUNKS = (512, 512)


def _pool_kernel(x_hbm, w_hbm, b_hbm, o_hbm, x_v, b_v, o_v, w_vmem, sems):
    offs = [0, _CHUNKS[0]]

    def wcopy(i):
        return pltpu.make_async_copy(
            w_hbm.at[pl.ds(offs[i], _CHUNKS[i]), :],
            w_vmem.at[pl.ds(offs[i], _CHUNKS[i]), :],
            sems.at[i],
        )

    def xcopy():
        return pltpu.make_async_copy(
            x_hbm.at[:, pl.ds(0, 8), :], x_v, sems.at[2]
        )

    def bcopy():
        return pltpu.make_async_copy(b_hbm, b_v, sems.at[3])

    xcopy().start()
    bcopy().start()
    wcopy(0).start()
    wcopy(1).start()

    xcopy().wait()
    bcopy().wait()
    x = x_v[:, 0, :]

    for i in range(2):
        wcopy(i).wait()
        w = w_vmem[pl.ds(offs[i], _CHUNKS[i]), :]
        acc = jax.lax.dot_general(
            x, w,
            dimension_numbers=(((1,), (1,)), ((), ())),
            preferred_element_type=jnp.float32,
        )
        o_v[:, pl.ds(offs[i], _CHUNKS[i])] = jnp.tanh(
            acc + b_v[:, pl.ds(offs[i], _CHUNKS[i])]
        )

    ocopy = pltpu.make_async_copy(o_v, o_hbm, sems.at[4])
    ocopy.start()
    ocopy.wait()


@jax.jit
def kernel(hidden_states, W, b):
    B, _, H = hidden_states.shape
    O = W.shape[0]

    b2 = b.reshape(1, O)

    out = pl.pallas_call(
        _pool_kernel,
        in_specs=[
            pl.BlockSpec(memory_space=pl.ANY),
            pl.BlockSpec(memory_space=pl.ANY),
            pl.BlockSpec(memory_space=pl.ANY),
        ],
        out_specs=pl.BlockSpec(memory_space=pl.ANY),
        out_shape=jax.ShapeDtypeStruct((B, O), jnp.float32),
        scratch_shapes=[
            pltpu.VMEM((B, 8, H), jnp.float32),
            pltpu.VMEM((1, O), jnp.float32),
            pltpu.VMEM((B, O), jnp.float32),
            pltpu.VMEM((O, H), jnp.float32),
            pltpu.SemaphoreType.DMA((5,)),
        ],
    )(hidden_states, W, b2)
    return out
